# baseline (speedup 1.0000x reference)
import functools

import jax
import jax.numpy as jnp
from jax import lax
from jax.experimental import pallas as pl
from jax.experimental.pallas import tpu as pltpu
from jax.experimental.pallas import tpu_sc as plsc

N_DOMAIN = 26
VOCAB = 100000
DIM = 64
BATCH = 16384
B_TOTAL = BATCH * N_DOMAIN
NW = 32
B_PER_W = B_TOTAL // NW
ROWS = 128
NG = B_PER_W // ROWS
LANES = 16

_mesh = plsc.VectorSubcoreMesh(core_axis_name="c", subcore_axis_name="s")


@functools.partial(
    pl.kernel,
    mesh=_mesh,
    compiler_params=pltpu.CompilerParams(use_tc_tiling_on_sc=False),
    out_type=jax.ShapeDtypeStruct((B_TOTAL, DIM), jnp.float32),
    scratch_types=[
        pltpu.VMEM((NG, ROWS), jnp.int32),
        pltpu.VMEM((ROWS, DIM), jnp.float32),
        pltpu.SemaphoreType.DMA,
    ],
)
def _gather_kernel(cond_hbm, table_hbm, out_hbm, idx_v, rows_v, sem):
    wid = lax.axis_index("s") * 2 + lax.axis_index("c")
    base = wid * B_PER_W
    pltpu.sync_copy(cond_hbm.at[wid], idx_v)

    def chunk(j, carry):
        for k in range(ROWS // LANES):
            p = base + j * ROWS + k * LANES + lax.iota(jnp.int32, LANES)
            off = (p % N_DOMAIN) * VOCAB
            idx_v[j, pl.ds(k * LANES, LANES)] = (
                idx_v[j, pl.ds(k * LANES, LANES)] + off
            )
        pltpu.async_copy(table_hbm.at[idx_v.at[j]], rows_v, sem).wait()
        pltpu.sync_copy(rows_v, out_hbm.at[pl.ds(base + j * ROWS, ROWS)])
        return carry

    lax.fori_loop(0, NG, chunk, 0)


def kernel(conditions, tables):
    cond_flat = conditions.astype(jnp.int32).reshape(NW, NG, ROWS)
    table_flat = tables.reshape(N_DOMAIN * VOCAB, DIM)
    out = _gather_kernel(cond_flat, table_flat)
    return out.reshape(BATCH, N_DOMAIN, 8, 8)


# speedup vs baseline: 1.0253x; 1.0253x over previous
import functools

import jax
import jax.numpy as jnp
from jax import lax
from jax.experimental import pallas as pl
from jax.experimental.pallas import tpu as pltpu
from jax.experimental.pallas import tpu_sc as plsc

N_DOMAIN = 26
VOCAB = 100000
DIM = 64
BATCH = 16384
B_TOTAL = BATCH * N_DOMAIN
NW = 32
B_PER_W = B_TOTAL // NW
ROWS = 256
NG = B_PER_W // ROWS
NBUF = 4
K = 2
GROUPS = NG // NBUF
LANES = 16

assert NG % NBUF == 0 and GROUPS >= 2 and K <= NBUF

_mesh = plsc.VectorSubcoreMesh(core_axis_name="c", subcore_axis_name="s")


@functools.partial(
    pl.kernel,
    mesh=_mesh,
    compiler_params=pltpu.CompilerParams(use_tc_tiling_on_sc=False),
    out_type=jax.ShapeDtypeStruct((B_TOTAL, DIM), jnp.float32),
    scratch_types=[
        pltpu.VMEM((NG, ROWS), jnp.int32),
        pltpu.VMEM((NBUF, ROWS, DIM), jnp.float32),
        pltpu.SemaphoreType.DMA((NBUF,)),
        pltpu.SemaphoreType.DMA((NBUF,)),
    ],
)
def _gather_kernel(cond_hbm, table_hbm, out_hbm, idx_v, rows_v, gsems, osems):
    wid = lax.axis_index("s") * 2 + lax.axis_index("c")
    base = wid * B_PER_W
    pltpu.sync_copy(cond_hbm.at[wid], idx_v)

    def compute_idx(j):
        for k in range(ROWS // LANES):
            p = base + j * ROWS + k * LANES + lax.iota(jnp.int32, LANES)
            off = (p % N_DOMAIN) * VOCAB
            idx_v[j, pl.ds(k * LANES, LANES)] = (
                idx_v[j, pl.ds(k * LANES, LANES)] + off
            )

    def start_gather(j, slot):
        pltpu.async_copy(table_hbm.at[idx_v.at[j]], rows_v.at[slot],
                         gsems.at[slot])

    def wait_gather(j, slot):
        pltpu.make_async_copy(table_hbm.at[idx_v.at[j]], rows_v.at[slot],
                              gsems.at[slot]).wait()

    def start_out(j, slot):
        pltpu.async_copy(rows_v.at[slot],
                         out_hbm.at[pl.ds(base + j * ROWS, ROWS)],
                         osems.at[slot])

    def wait_out(j, slot):
        pltpu.make_async_copy(rows_v.at[slot],
                              out_hbm.at[pl.ds(base + j * ROWS, ROWS)],
                              osems.at[slot]).wait()

    def emit_iter(j, b, do_wait_out, do_prefetch):
        wait_gather(j, b)
        start_out(j, b)
        slot_pf = (b + K) % NBUF
        if do_wait_out:
            wait_out(j - (NBUF - K), slot_pf)
        if do_prefetch:
            compute_idx(j + K)
            start_gather(j + K, slot_pf)

    for b in range(K):
        compute_idx(b)
        start_gather(b, b)

    for b in range(NBUF):
        emit_iter(b, b, do_wait_out=(b >= NBUF - K), do_prefetch=True)

    def group(g, carry):
        for b in range(NBUF):
            emit_iter(g * NBUF + b, b, do_wait_out=True, do_prefetch=True)
        return carry

    lax.fori_loop(1, GROUPS - 1, group, 0)

    for b in range(NBUF):
        j = (GROUPS - 1) * NBUF + b
        emit_iter(j, b, do_wait_out=True, do_prefetch=(b < NBUF - K))

    for b in range(NBUF - K, NBUF):
        j = (GROUPS - 1) * NBUF + b
        wait_out(j, b)


def kernel(conditions, tables):
    cond_flat = conditions.astype(jnp.int32).reshape(NW, NG, ROWS)
    table_flat = tables.reshape(N_DOMAIN * VOCAB, DIM)
    out = _gather_kernel(cond_flat, table_flat)
    return out.reshape(BATCH, N_DOMAIN, 8, 8)


# speedup vs baseline: 3.8412x; 3.7466x over previous
import functools

import jax
import jax.numpy as jnp
from jax import lax
from jax.experimental import pallas as pl
from jax.experimental.pallas import tpu as pltpu
from jax.experimental.pallas import tpu_sc as plsc

N_DOMAIN = 26
VOCAB = 100000
DIM = 64
BATCH = 16384
NW = 32
N_ROWS = N_DOMAIN * DIM
R_PER_W = N_ROWS // NW
OCHUNK = 4096
NOB = BATCH // OCHUNK
LANES = 16

_mesh = plsc.VectorSubcoreMesh(core_axis_name="c", subcore_axis_name="s")


@functools.partial(
    pl.kernel,
    mesh=_mesh,
    compiler_params=pltpu.CompilerParams(needs_layout_passes=False),
    out_type=jax.ShapeDtypeStruct((N_DOMAIN, DIM, BATCH), jnp.float32),
    scratch_types=[
        pltpu.VMEM((VOCAB,), jnp.float32),
        pltpu.VMEM((BATCH,), jnp.int32),
        pltpu.VMEM((2, OCHUNK), jnp.float32),
        pltpu.SemaphoreType.DMA,
        pltpu.SemaphoreType.DMA((2,)),
    ],
)
def _gather_kernel(cond_hbm, tables_hbm, out_hbm, row_v, idx_v, obuf, lsem,
                   osems):
    wid = lax.axis_index("s") * 2 + lax.axis_index("c")
    r0 = wid * R_PER_W
    i0 = r0 // DIM

    def load_idx(i):
        pltpu.async_copy(cond_hbm.at[i], idx_v, lsem).wait()

    def do_row(r, carry):
        i = r // DIM
        d = r % DIM
        pltpu.async_copy(tables_hbm.at[i, d], row_v, lsem).wait()
        for c in range(NOB):
            slot = c % 2
            @pl.when(jnp.logical_or(r > r0, c >= 2))
            def _():
                pltpu.make_async_copy(
                    obuf.at[slot],
                    out_hbm.at[i, d, pl.ds(0, OCHUNK)],
                    osems.at[slot],
                ).wait()

            def gather16(k, _):
                idxv = idx_v[pl.ds(c * OCHUNK + k * LANES, LANES)]
                vals = plsc.load_gather(row_v, [idxv])
                obuf[slot, pl.ds(k * LANES, LANES)] = vals
                return _

            lax.fori_loop(0, OCHUNK // LANES, gather16, 0, unroll=8)
            pltpu.async_copy(
                obuf.at[slot],
                out_hbm.at[i, d, pl.ds(c * OCHUNK, OCHUNK)],
                osems.at[slot],
            )
        return carry

    seg_end = jnp.minimum((i0 + 1) * DIM, r0 + R_PER_W)
    load_idx(i0)
    lax.fori_loop(r0, seg_end, do_row, 0)

    @pl.when(seg_end < r0 + R_PER_W)
    def _():
        load_idx(i0 + 1)
        lax.fori_loop(seg_end, r0 + R_PER_W, do_row, 0)

    for slot in range(2):
        pltpu.make_async_copy(
            obuf.at[slot],
            out_hbm.at[0, 0, pl.ds(0, OCHUNK)],
            osems.at[slot],
        ).wait()


def kernel(conditions, tables):
    cond_t = conditions.astype(jnp.int32).T
    tables_t = jnp.transpose(tables, (0, 2, 1))
    out = _gather_kernel(cond_t, tables_t)
    out = out.reshape(N_DOMAIN, 8, 8, BATCH)
    return jnp.transpose(out, (3, 0, 1, 2))
